# baseline (speedup 1.0000x reference)
import functools

import jax
import jax.numpy as jnp
from jax import lax
from jax.experimental import pallas as pl
from jax.experimental.pallas import tpu as pltpu
from jax.experimental.pallas import tpu_sc as plsc


_BETA = 0.25


def _normcb_body(cb_ref, out_ref):
    cb = cb_ref[...]
    n = jnp.sqrt(jnp.sum(cb * cb, axis=1, keepdims=True))
    out_ref[...] = cb / jnp.maximum(n, 1e-12)


def _make_argmin_body(bm, k):
    sub = 128
    kc = 2048

    def body(x_ref, cb_ref, ids_ref, w16_ref, acc_ref):
        i = pl.program_id(0)
        xb = x_ref[...]
        ssq = jnp.sum(xb * xb, axis=1, keepdims=True)
        nrm = jnp.sqrt(ssq)
        xn = xb / jnp.maximum(nrm, 1e-12)

        lane = lax.broadcasted_iota(jnp.int32, (1, 128), 1)
        acc_p = jnp.float32(0.0)
        for s in range(bm // sub):
            xns = xn[s * sub:(s + 1) * sub, :]
            dmin = jnp.full((sub, 128), jnp.inf, jnp.float32)
            gcol = jnp.zeros((sub, 128), jnp.int32)
            for ck in range(k // kc):
                sim = lax.dot_general(
                    xns, cb_ref[:, pl.ds(ck * kc, kc)],
                    (((1,), (0,)), ((), ())),
                    preferred_element_type=jnp.float32)
                for c in range(kc // 128):
                    dc = 1.0 - sim[:, c * 128:(c + 1) * 128]
                    upd = dc < dmin
                    dmin = jnp.minimum(dc, dmin)
                    gcol = jnp.where(
                        upd, jnp.int32(ck * (kc // 128) + c), gcol)

            mind = jnp.min(dmin, axis=1)
            kidx = gcol * 128 + lane
            best_i = jnp.min(
                jnp.where(dmin == mind[:, None], kidx, jnp.int32(k)),
                axis=1)

            msim = 1.0 - mind
            w = jnp.maximum(nrm[s * sub:(s + 1) * sub, 0] * msim, 1e-6)
            ids_ref[0, 0, s * sub:(s + 1) * sub] = best_i
            w16_ref[s * sub:(s + 1) * sub, :] = jnp.broadcast_to(
                w[:, None], (sub, 16))
            acc_p += jnp.sum(mind)

        @pl.when(i == 0)
        def _():
            acc_ref[...] = jnp.zeros((1, 1), jnp.float32)

        acc_ref[...] += jnp.reshape(acc_p, (1, 1))

    return body


def _sc_gather_scale(ids_flat, w16, norm_cb, m, d):
    info = plsc.get_sparse_core_info()
    nc, ns, nl = info.num_cores, info.num_subcores, info.num_lanes
    nw = nc * ns
    bpw = m // nw
    ch = 128
    nchunks = bpw // ch

    @functools.partial(
        pl.kernel,
        out_type=jax.ShapeDtypeStruct((m, d), jnp.float32),
        mesh=plsc.VectorSubcoreMesh(core_axis_name="c", subcore_axis_name="s"),
        scratch_types=[
            pltpu.VMEM((ch,), jnp.int32),
            pltpu.VMEM((ch, 16), jnp.float32),
            pltpu.VMEM((ch, d), jnp.float32),
            pltpu.SemaphoreType.DMA,
        ],
    )
    def gather_kernel(ids_hbm, w16_hbm, tab_hbm, out_hbm, idx_v, w_v, rows_v, sem):
        wid = lax.axis_index("s") * nc + lax.axis_index("c")
        base = wid * bpw

        def chunk_body(cidx, carry):
            off = base + cidx * ch
            pltpu.sync_copy(ids_hbm.at[pl.ds(off, ch)], idx_v)
            pltpu.sync_copy(w16_hbm.at[pl.ds(off, ch)], w_v)
            pltpu.async_copy(tab_hbm.at[idx_v], rows_v, sem).wait()

            def row_body(r, carry2):
                wvec = w_v[r, :]
                for g in range(d // nl):
                    sl = pl.ds(g * nl, nl)
                    rows_v[r, sl] = rows_v[r, sl] * wvec
                return carry2

            lax.fori_loop(0, ch, row_body, 0, unroll=8)
            pltpu.sync_copy(rows_v, out_hbm.at[pl.ds(off, ch)])
            return carry

        lax.fori_loop(0, nchunks, chunk_body, 0, unroll=False)

    return gather_kernel(ids_flat, w16, norm_cb)


def kernel(x, codebook):
    b, t, d = x.shape
    k = codebook.shape[0]
    m = b * t
    bm = 2048
    mt = m // bm
    xf = x.reshape(m, d)

    norm_cb = pl.pallas_call(
        _normcb_body,
        out_shape=jax.ShapeDtypeStruct((k, d), jnp.float32),
    )(codebook)

    cbt = norm_cb.T

    nh = 2
    mh = m // nh
    mth = mh // bm
    argmin_call = pl.pallas_call(
        _make_argmin_body(bm, k),
        grid=(mth,),
        in_specs=[
            pl.BlockSpec((bm, d), lambda i: (i, 0)),
            pl.BlockSpec((d, k), lambda i: (0, 0)),
        ],
        out_specs=[
            pl.BlockSpec((1, 1, bm), lambda i: (i, 0, 0)),
            pl.BlockSpec((bm, 16), lambda i: (i, 0)),
            pl.BlockSpec((1, 1), lambda i: (0, 0)),
        ],
        out_shape=[
            jax.ShapeDtypeStruct((mth, 1, bm), jnp.int32),
            jax.ShapeDtypeStruct((mh, 16), jnp.float32),
            jax.ShapeDtypeStruct((1, 1), jnp.float32),
        ],
        compiler_params=pltpu.CompilerParams(
            dimension_semantics=("arbitrary",)),
    )

    ids_hs, wemb_hs, acc_sum = [], [], jnp.float32(0.0)
    for h in range(nh):
        ids3, w16, acc = argmin_call(xf[h * mh:(h + 1) * mh], cbt)
        ids_h = ids3.reshape(mh)
        ids_hs.append(ids_h)
        wemb_hs.append(_sc_gather_scale(ids_h, w16, norm_cb, mh, d))
        acc_sum = acc_sum + acc[0, 0]

    ids = jnp.concatenate(ids_hs).reshape(b, t)
    wemb = jnp.concatenate(wemb_hs)
    loss = (1.0 + _BETA) * acc_sum / m
    return (ids, wemb.reshape(b, t, d), loss)


# speedup vs baseline: 1.1549x; 1.1549x over previous
import functools

import jax
import jax.numpy as jnp
from jax import lax
from jax.experimental import pallas as pl
from jax.experimental.pallas import tpu as pltpu
from jax.experimental.pallas import tpu_sc as plsc


_BETA = 0.25


def _normcb_body(cb_ref, out_ref):
    cb = cb_ref[...]
    n = jnp.sqrt(jnp.sum(cb * cb, axis=1, keepdims=True))
    out_ref[...] = cb / jnp.maximum(n, 1e-12)


def _make_argmin_body(bm, k):
    sub = 128
    kc = 2048

    def body(x_ref, cb_ref, ids_ref, w16_ref, acc_ref):
        i = pl.program_id(0)
        xb = x_ref[...]
        ssq = jnp.sum(xb * xb, axis=1, keepdims=True)
        nrm = jnp.sqrt(ssq)
        xn = xb / jnp.maximum(nrm, 1e-12)

        lane = lax.broadcasted_iota(jnp.int32, (1, 128), 1)
        acc_p = jnp.float32(0.0)
        for s in range(bm // sub):
            xns = xn[s * sub:(s + 1) * sub, :]
            dmin = jnp.full((sub, 128), jnp.inf, jnp.float32)
            gcol = jnp.zeros((sub, 128), jnp.int32)
            for ck in range(k // kc):
                sim = lax.dot_general(
                    xns, cb_ref[:, pl.ds(ck * kc, kc)],
                    (((1,), (0,)), ((), ())),
                    preferred_element_type=jnp.float32)
                for c in range(kc // 128):
                    dc = 1.0 - sim[:, c * 128:(c + 1) * 128]
                    upd = dc < dmin
                    dmin = jnp.minimum(dc, dmin)
                    gcol = jnp.where(
                        upd, jnp.int32(ck * (kc // 128) + c), gcol)

            mind = jnp.min(dmin, axis=1)
            kidx = gcol * 128 + lane
            best_i = jnp.min(
                jnp.where(dmin == mind[:, None], kidx, jnp.int32(k)),
                axis=1)

            msim = 1.0 - mind
            w = jnp.maximum(nrm[s * sub:(s + 1) * sub, 0] * msim, 1e-6)
            ids_ref[0, 0, s * sub:(s + 1) * sub] = best_i
            w16_ref[s * sub:(s + 1) * sub, :] = jnp.broadcast_to(
                w[:, None], (sub, 16))
            acc_p += jnp.sum(mind)

        @pl.when(i == 0)
        def _():
            acc_ref[...] = jnp.zeros((1, 1), jnp.float32)

        acc_ref[...] += jnp.reshape(acc_p, (1, 1))

    return body


def _sc_gather_scale(ids_flat, w16, norm_cb, m, d):
    info = plsc.get_sparse_core_info()
    nc, ns, nl = info.num_cores, info.num_subcores, info.num_lanes
    nw = nc * ns
    bpw = m // nw
    ch = 128
    nchunks = bpw // ch

    @functools.partial(
        pl.kernel,
        out_type=jax.ShapeDtypeStruct((m, d), jnp.float32),
        mesh=plsc.VectorSubcoreMesh(core_axis_name="c", subcore_axis_name="s"),
        scratch_types=[
            pltpu.VMEM((2, ch), jnp.int32),
            pltpu.VMEM((2, ch, 16), jnp.float32),
            pltpu.VMEM((2, ch, d), jnp.float32),
            pltpu.SemaphoreType.DMA,
            pltpu.SemaphoreType.DMA,
            pltpu.SemaphoreType.DMA,
            pltpu.SemaphoreType.DMA,
            pltpu.SemaphoreType.DMA,
            pltpu.SemaphoreType.DMA,
        ],
    )
    def gather_kernel(ids_hbm, w16_hbm, tab_hbm, out_hbm, idx_v, w_v, rows_v,
                      si0, si1, sg0, sg1, so0, so1):
        wid = lax.axis_index("s") * nc + lax.axis_index("c")
        base = wid * bpw
        sis = (si0, si1)
        sgs = (sg0, sg1)
        sos = (so0, so1)

        def start_fetch(cidx, buf):
            off = base + cidx * ch
            pltpu.async_copy(ids_hbm.at[pl.ds(off, ch)], idx_v.at[buf],
                             sis[buf])
            pltpu.async_copy(w16_hbm.at[pl.ds(off, ch)], w_v.at[buf],
                             sis[buf])

        def start_gather(buf):
            pltpu.async_copy(tab_hbm.at[idx_v.at[buf]], rows_v.at[buf],
                             sgs[buf])

        def drain_fetch(cidx, buf):
            pltpu.make_async_copy(ids_hbm.at[pl.ds(base, ch)],
                                  idx_v.at[buf], sis[buf]).wait()
            pltpu.make_async_copy(w16_hbm.at[pl.ds(base, ch)],
                                  w_v.at[buf], sis[buf]).wait()

        def scale(buf):
            def row_body(r, carry2):
                wvec = w_v[buf, r, :]
                for g in range(d // nl):
                    sl = pl.ds(g * nl, nl)
                    rows_v[buf, r, sl] = rows_v[buf, r, sl] * wvec
                return carry2

            lax.fori_loop(0, ch, row_body, 0, unroll=8)

        start_fetch(0, 0)
        drain_fetch(0, 0)
        start_gather(0)
        for cidx in range(nchunks):
            buf = cidx % 2
            nbuf = 1 - buf
            if cidx + 1 < nchunks:
                start_fetch(cidx + 1, nbuf)
                drain_fetch(cidx + 1, nbuf)
                if cidx >= 1:
                    pltpu.make_async_copy(
                        rows_v.at[nbuf],
                        out_hbm.at[pl.ds(base, ch)], sos[nbuf]).wait()
                start_gather(nbuf)
            pltpu.make_async_copy(
                tab_hbm.at[idx_v.at[buf]], rows_v.at[buf], sgs[buf]).wait()
            scale(buf)
            off = base + cidx * ch
            pltpu.async_copy(rows_v.at[buf], out_hbm.at[pl.ds(off, ch)],
                             sos[buf])
        pltpu.make_async_copy(
            rows_v.at[(nchunks - 1) % 2],
            out_hbm.at[pl.ds(base, ch)], sos[(nchunks - 1) % 2]).wait()
        pltpu.make_async_copy(
            rows_v.at[nchunks % 2],
            out_hbm.at[pl.ds(base, ch)], sos[nchunks % 2]).wait()

    return gather_kernel(ids_flat, w16, norm_cb)


def kernel(x, codebook):
    b, t, d = x.shape
    k = codebook.shape[0]
    m = b * t
    bm = 2048
    mt = m // bm
    xf = x.reshape(m, d)

    norm_cb = pl.pallas_call(
        _normcb_body,
        out_shape=jax.ShapeDtypeStruct((k, d), jnp.float32),
    )(codebook)

    ids3, w16, acc = pl.pallas_call(
        _make_argmin_body(bm, k),
        grid=(mt,),
        in_specs=[
            pl.BlockSpec((bm, d), lambda i: (i, 0)),
            pl.BlockSpec((d, k), lambda i: (0, 0)),
        ],
        out_specs=[
            pl.BlockSpec((1, 1, bm), lambda i: (i, 0, 0)),
            pl.BlockSpec((bm, 16), lambda i: (i, 0)),
            pl.BlockSpec((1, 1), lambda i: (0, 0)),
        ],
        out_shape=[
            jax.ShapeDtypeStruct((mt, 1, bm), jnp.int32),
            jax.ShapeDtypeStruct((m, 16), jnp.float32),
            jax.ShapeDtypeStruct((1, 1), jnp.float32),
        ],
        compiler_params=pltpu.CompilerParams(
            dimension_semantics=("arbitrary",)),
    )(xf, norm_cb.T)

    ids_flat = ids3.reshape(m)

    wemb = _sc_gather_scale(ids_flat, w16, norm_cb, m, d)

    ids = ids_flat.reshape(b, t)
    loss = (1.0 + _BETA) * acc[0, 0] / m
    return (ids, wemb.reshape(b, t, d), loss)


# speedup vs baseline: 1.1784x; 1.0204x over previous
import functools

import jax
import jax.numpy as jnp
from jax import lax
from jax.experimental import pallas as pl
from jax.experimental.pallas import tpu as pltpu
from jax.experimental.pallas import tpu_sc as plsc


_BETA = 0.25


def _normcb_body(cb_ref, out_ref, outt_ref):
    cb = cb_ref[...]
    n = jnp.sqrt(jnp.sum(cb * cb, axis=1, keepdims=True))
    ncb = cb / jnp.maximum(n, 1e-12)
    out_ref[...] = ncb
    outt_ref[...] = ncb.T


def _make_argmin_body(bm, k):
    sub = 128
    kc = 2048

    def body(x_ref, cb_ref, ids_ref, w16_ref, acc_ref):
        i = pl.program_id(0)
        xb = x_ref[...]
        ssq = jnp.sum(xb * xb, axis=1, keepdims=True)
        nrm = jnp.sqrt(ssq)
        xn = xb / jnp.maximum(nrm, 1e-12)

        lane = lax.broadcasted_iota(jnp.int32, (1, 128), 1)
        acc_p = jnp.float32(0.0)
        for s in range(bm // sub):
            xns = xn[s * sub:(s + 1) * sub, :]
            dmin = jnp.full((sub, 128), jnp.inf, jnp.float32)
            gcol = jnp.zeros((sub, 128), jnp.int32)
            for ck in range(k // kc):
                sim = lax.dot_general(
                    xns, cb_ref[:, pl.ds(ck * kc, kc)],
                    (((1,), (0,)), ((), ())),
                    preferred_element_type=jnp.float32)
                for c in range(kc // 128):
                    dc = 1.0 - sim[:, c * 128:(c + 1) * 128]
                    upd = dc < dmin
                    dmin = jnp.minimum(dc, dmin)
                    gcol = jnp.where(
                        upd, jnp.int32(ck * (kc // 128) + c), gcol)

            mind = jnp.min(dmin, axis=1)
            kidx = gcol * 128 + lane
            best_i = jnp.min(
                jnp.where(dmin == mind[:, None], kidx, jnp.int32(k)),
                axis=1)

            msim = 1.0 - mind
            w = jnp.maximum(nrm[s * sub:(s + 1) * sub, 0] * msim, 1e-6)
            ids_ref[0, 0, s * sub:(s + 1) * sub] = best_i
            w16_ref[s * sub:(s + 1) * sub, :] = jnp.broadcast_to(
                w[:, None], (sub, 16))
            acc_p += jnp.sum(mind)

        @pl.when(i == 0)
        def _():
            acc_ref[...] = jnp.zeros((1, 1), jnp.float32)

        acc_ref[...] += jnp.reshape(acc_p, (1, 1))

    return body


def _sc_gather_scale(ids_flat, w16, norm_cb, m, d):
    info = plsc.get_sparse_core_info()
    nc, ns, nl = info.num_cores, info.num_subcores, info.num_lanes
    nw = nc * ns
    bpw = m // nw
    ch = 128
    nchunks = bpw // ch

    @functools.partial(
        pl.kernel,
        out_type=jax.ShapeDtypeStruct((m, d), jnp.float32),
        mesh=plsc.VectorSubcoreMesh(core_axis_name="c", subcore_axis_name="s"),
        scratch_types=[
            pltpu.VMEM((2, ch), jnp.int32),
            pltpu.VMEM((2, ch, 16), jnp.float32),
            pltpu.VMEM((2, ch, d), jnp.float32),
            pltpu.SemaphoreType.DMA,
            pltpu.SemaphoreType.DMA,
            pltpu.SemaphoreType.DMA,
            pltpu.SemaphoreType.DMA,
            pltpu.SemaphoreType.DMA,
            pltpu.SemaphoreType.DMA,
        ],
    )
    def gather_kernel(ids_hbm, w16_hbm, tab_hbm, out_hbm, idx_v, w_v, rows_v,
                      si0, si1, sg0, sg1, so0, so1):
        wid = lax.axis_index("s") * nc + lax.axis_index("c")
        base = wid * bpw
        sis = (si0, si1)
        sgs = (sg0, sg1)
        sos = (so0, so1)

        def start_fetch(cidx, buf):
            off = base + cidx * ch
            pltpu.async_copy(ids_hbm.at[pl.ds(off, ch)], idx_v.at[buf],
                             sis[buf])
            pltpu.async_copy(w16_hbm.at[pl.ds(off, ch)], w_v.at[buf],
                             sis[buf])

        def start_gather(buf):
            pltpu.async_copy(tab_hbm.at[idx_v.at[buf]], rows_v.at[buf],
                             sgs[buf])

        def drain_fetch(cidx, buf):
            pltpu.make_async_copy(ids_hbm.at[pl.ds(base, ch)],
                                  idx_v.at[buf], sis[buf]).wait()
            pltpu.make_async_copy(w16_hbm.at[pl.ds(base, ch)],
                                  w_v.at[buf], sis[buf]).wait()

        def scale(buf):
            def row_body(r, carry2):
                wvec = w_v[buf, r, :]
                for g in range(d // nl):
                    sl = pl.ds(g * nl, nl)
                    rows_v[buf, r, sl] = rows_v[buf, r, sl] * wvec
                return carry2

            lax.fori_loop(0, ch, row_body, 0, unroll=8)

        start_fetch(0, 0)
        drain_fetch(0, 0)
        start_gather(0)
        for cidx in range(nchunks):
            buf = cidx % 2
            nbuf = 1 - buf
            if cidx + 1 < nchunks:
                start_fetch(cidx + 1, nbuf)
                drain_fetch(cidx + 1, nbuf)
                if cidx >= 1:
                    pltpu.make_async_copy(
                        rows_v.at[nbuf],
                        out_hbm.at[pl.ds(base, ch)], sos[nbuf]).wait()
                start_gather(nbuf)
            pltpu.make_async_copy(
                tab_hbm.at[idx_v.at[buf]], rows_v.at[buf], sgs[buf]).wait()
            scale(buf)
            off = base + cidx * ch
            pltpu.async_copy(rows_v.at[buf], out_hbm.at[pl.ds(off, ch)],
                             sos[buf])
        pltpu.make_async_copy(
            rows_v.at[(nchunks - 1) % 2],
            out_hbm.at[pl.ds(base, ch)], sos[(nchunks - 1) % 2]).wait()
        pltpu.make_async_copy(
            rows_v.at[nchunks % 2],
            out_hbm.at[pl.ds(base, ch)], sos[nchunks % 2]).wait()

    return gather_kernel(ids_flat, w16, norm_cb)


def kernel(x, codebook):
    b, t, d = x.shape
    k = codebook.shape[0]
    m = b * t
    bm = 2048
    mt = m // bm
    xf = x.reshape(m, d)

    norm_cb, cbt = pl.pallas_call(
        _normcb_body,
        out_shape=[
            jax.ShapeDtypeStruct((k, d), jnp.float32),
            jax.ShapeDtypeStruct((d, k), jnp.float32),
        ],
    )(codebook)

    ids3, w16, acc = pl.pallas_call(
        _make_argmin_body(bm, k),
        grid=(mt,),
        in_specs=[
            pl.BlockSpec((bm, d), lambda i: (i, 0)),
            pl.BlockSpec((d, k), lambda i: (0, 0)),
        ],
        out_specs=[
            pl.BlockSpec((1, 1, bm), lambda i: (i, 0, 0)),
            pl.BlockSpec((bm, 16), lambda i: (i, 0)),
            pl.BlockSpec((1, 1), lambda i: (0, 0)),
        ],
        out_shape=[
            jax.ShapeDtypeStruct((mt, 1, bm), jnp.int32),
            jax.ShapeDtypeStruct((m, 16), jnp.float32),
            jax.ShapeDtypeStruct((1, 1), jnp.float32),
        ],
        compiler_params=pltpu.CompilerParams(
            dimension_semantics=("arbitrary",)),
    )(xf, cbt)

    ids_flat = ids3.reshape(m)

    wemb = _sc_gather_scale(ids_flat, w16, norm_cb, m, d)

    ids = ids_flat.reshape(b, t)
    loss = (1.0 + _BETA) * acc[0, 0] / m
    return (ids, wemb.reshape(b, t, d), loss)
